# baseline (speedup 1.0000x reference)
import functools

import jax
import jax.numpy as jnp
from jax import lax
from jax.experimental import pallas as pl
from jax.experimental.pallas import tpu as pltpu
from jax.experimental.pallas import tpu_sc as plsc

_N = 10000
_E = 320000
_D = 128
_G = 64
_NC = 10
_NUM_GCN = 4

_NTILES = 32
_IW = 128
_CW = 128
_NBUF = 2
_PHROWS = 40
_NPH = 2
_IROWS_T = _PHROWS * _NPH
_NIROWS = _NTILES * _IROWS_T
_EP = _NIROWS * _IW
_CPP = _PHROWS
_RPT = 640
_NPAD = _RPT * 16
_NACC = _NPAD + 8
_ROWBLK = 1000
_NBLK = _N // _ROWBLK


def _sc_body(m_hbm, src_hbm, dst_hbm, out_hbm, src_v, dst_v,
             r0, r1, acc, s0, s1):
    cid = lax.axis_index("c")
    sid = lax.axis_index("s")
    wid = cid * 16 + sid
    rbufs = (r0, r1)
    sems = (s0, s1)

    def _zrow(i, _):
        for j in range(8):
            r0[i, pl.ds(j * 16, 16)] = jnp.zeros((16,), jnp.float32)
        return 0

    lax.fori_loop(0, _CW, _zrow, 0)
    for t in range(_RPT // _CW):
        pltpu.sync_copy(r0, acc.at[pl.ds(sid * _RPT + t * _CW, _CW)])
    plsc.subcore_barrier()

    def _sidx(g, b):
        return src_v.at[_NBUF * g + b]

    def _didx(g, b):
        return dst_v.at[_NBUF * g + b]

    def _phase(p):
        rstart = wid * _IROWS_T + p * _PHROWS
        pltpu.sync_copy(src_hbm.at[pl.ds(rstart, _PHROWS)], src_v)
        pltpu.sync_copy(dst_hbm.at[pl.ds(rstart, _PHROWS)], dst_v)
        for b in range(_NBUF):
            pltpu.async_copy(m_hbm.at[_sidx(0, b)], rbufs[b], sems[b])

        def _group(g, _):
            for b in range(_NBUF):
                pltpu.make_async_copy(
                    m_hbm.at[_sidx(g, b)], rbufs[b], sems[b]).wait()
                pltpu.sync_copy(rbufs[b], acc.at[_didx(g, b)], add=True)
                pltpu.async_copy(m_hbm.at[_sidx(g + 1, b)], rbufs[b], sems[b])
            return 0

        lax.fori_loop(0, _CPP // _NBUF - 1, _group, 0)
        g_last = _CPP // _NBUF - 1
        for b in range(_NBUF):
            pltpu.make_async_copy(
                m_hbm.at[_sidx(g_last, b)], rbufs[b], sems[b]).wait()
            pltpu.sync_copy(rbufs[b], acc.at[_didx(g_last, b)], add=True)

    for p in range(_NPH):
        _phase(p)
    plsc.subcore_barrier()

    pltpu.sync_copy(
        acc.at[pl.ds(sid * _RPT, _RPT)],
        out_hbm.at[cid, pl.ds(sid * _RPT, _RPT)],
    )


@functools.lru_cache(maxsize=1)
def _sc_scatter_fn():
    return functools.partial(
        pl.kernel,
        out_type=jax.ShapeDtypeStruct((2, _NPAD, _D), jnp.float32),
        mesh=plsc.VectorSubcoreMesh(core_axis_name="c", subcore_axis_name="s"),
        scratch_types=[
            pltpu.VMEM((_PHROWS, _IW), jnp.int32),
            pltpu.VMEM((_PHROWS, _IW), jnp.int32),
            pltpu.VMEM((_CW, _D), jnp.float32),
            pltpu.VMEM((_CW, _D), jnp.float32),
            pltpu.VMEM_SHARED((_NACC, _D), jnp.float32),
            pltpu.SemaphoreType.DMA,
            pltpu.SemaphoreType.DMA,
        ],
    )(_sc_body)


def _sc_scatter(m, src_p, dst_p):
    return _sc_scatter_fn()(m, src_p, dst_p)


def _embed_body(x_ref, we_ref, be_ref, w0_ref, o_ref):
    h = jnp.dot(x_ref[...], we_ref[...], preferred_element_type=jnp.float32)
    h = h + be_ref[...]
    o_ref[...] = jnp.dot(h, w0_ref[...], preferred_element_type=jnp.float32)


def _layer_body(p_ref, b_ref, w_ref, o_ref):
    h = jax.nn.relu(p_ref[0] + p_ref[1] + b_ref[...])
    o_ref[...] = jnp.dot(h, w_ref[...], preferred_element_type=jnp.float32)


def _final_body(p_ref, b_ref, pmat_ref, w0_ref, b0_ref, w1_ref, b1_ref,
                w2_ref, b2_ref, o_ref, pooled, counts):
    i = pl.program_id(0)

    @pl.when(i == 0)
    def _init():
        pooled[...] = jnp.zeros_like(pooled)
        counts[...] = jnp.zeros_like(counts)

    h = jax.nn.relu(p_ref[0] + p_ref[1] + b_ref[...])
    pmat_t = pmat_ref[...]
    dn = (((0,), (0,)), ((), ()))
    pooled[...] += lax.dot_general(
        pmat_t, h, dn, preferred_element_type=jnp.float32)
    counts[...] += lax.dot_general(
        pmat_t, jnp.ones((_ROWBLK, _D), jnp.float32), dn,
        preferred_element_type=jnp.float32)

    @pl.when(i == _NBLK - 1)
    def _finish():
        hg = pooled[...] / jnp.maximum(counts[...], 1.0)
        y = jax.nn.relu(
            jnp.dot(hg, w0_ref[...], preferred_element_type=jnp.float32)
            + b0_ref[...])
        y = jax.nn.relu(
            jnp.dot(y, w1_ref[...], preferred_element_type=jnp.float32)
            + b1_ref[...])
        o_ref[...] = (
            jnp.dot(y, w2_ref[...], preferred_element_type=jnp.float32)
            + b2_ref[...])


_full_w = pl.BlockSpec((_D, _D), lambda i: (0, 0))
_row_b = pl.BlockSpec((1, _D), lambda i: (0, 0))
_xblk = pl.BlockSpec((_ROWBLK, _D), lambda i: (i, 0))
_pblk = pl.BlockSpec((2, _ROWBLK, _D), lambda i: (0, i, 0))


def _tc_embed(x, w_emb, b_emb, w0):
    return pl.pallas_call(
        _embed_body,
        grid=(_NBLK,),
        in_specs=[_xblk, _full_w, _row_b, _full_w],
        out_specs=_xblk,
        out_shape=jax.ShapeDtypeStruct((_N, _D), jnp.float32),
    )(x, w_emb, b_emb, w0)


def _tc_layer(parts, b_prev, w):
    return pl.pallas_call(
        _layer_body,
        grid=(_NBLK,),
        in_specs=[_pblk, _row_b, _full_w],
        out_specs=_xblk,
        out_shape=jax.ShapeDtypeStruct((_N, _D), jnp.float32),
    )(parts, b_prev, w)


def _tc_final(parts, b_prev, pmat, w0, b0, w1, b1, w2, b2):
    return pl.pallas_call(
        _final_body,
        grid=(_NBLK,),
        in_specs=[
            _pblk,
            _row_b,
            pl.BlockSpec((_ROWBLK, _G), lambda i: (i, 0)),
            pl.BlockSpec((_D, _D // 2), lambda i: (0, 0)),
            pl.BlockSpec((1, _D // 2), lambda i: (0, 0)),
            pl.BlockSpec((_D // 2, _D // 4), lambda i: (0, 0)),
            pl.BlockSpec((1, _D // 4), lambda i: (0, 0)),
            pl.BlockSpec((_D // 4, _NC), lambda i: (0, 0)),
            pl.BlockSpec((1, _NC), lambda i: (0, 0)),
        ],
        out_specs=pl.BlockSpec((_G, _NC), lambda i: (0, 0)),
        out_shape=jax.ShapeDtypeStruct((_G, _NC), jnp.float32),
        scratch_shapes=[
            pltpu.VMEM((_G, _D), jnp.float32),
            pltpu.VMEM((_G, _D), jnp.float32),
        ],
    )(parts, b_prev, pmat, w0, b0, w1, b1, w2, b2)


def kernel(x, edge_index, batch, W_emb, b_emb, W_gcn, b_gcn,
           W_mlp0, b_mlp0, W_mlp1, b_mlp1, W_mlp2, b_mlp2):
    src = edge_index[0]
    dst = edge_index[1]
    pad = _EP - _E
    src_p = jnp.concatenate(
        [src, jnp.zeros((pad,), jnp.int32)]).reshape(_NIROWS, _IW)
    dst_p = jnp.concatenate(
        [dst, jnp.full((pad,), _NPAD, jnp.int32)]).reshape(_NIROWS, _IW)
    pmat = (batch[:, None] == jnp.arange(_G, dtype=batch.dtype)[None, :]
            ).astype(jnp.float32)

    b_emb2 = b_emb.reshape(1, _D)
    m = _tc_embed(x, W_emb, b_emb2, W_gcn[0])
    for l in range(_NUM_GCN):
        parts = _sc_scatter(m, src_p, dst_p)
        bl = b_gcn[l].reshape(1, _D)
        if l + 1 < _NUM_GCN:
            m = _tc_layer(parts, bl, W_gcn[l + 1])
        else:
            logits = _tc_final(parts, bl, pmat,
                               W_mlp0, b_mlp0.reshape(1, -1),
                               W_mlp1, b_mlp1.reshape(1, -1),
                               W_mlp2, b_mlp2.reshape(1, -1))
    return logits


# speedup vs baseline: 1.3243x; 1.3243x over previous
import functools

import jax
import jax.numpy as jnp
from jax import lax
from jax.experimental import pallas as pl
from jax.experimental.pallas import tpu as pltpu
from jax.experimental.pallas import tpu_sc as plsc

_N = 10000
_E = 320000
_D = 128
_G = 64
_NC = 10
_NUM_GCN = 4

_NTILES = 32
_CHUNK = 128
_NCHUNK = 80
_PHCHUNK = 40
_EPT = _CHUNK * _NCHUNK
_EP = _EPT * _NTILES
_RPT = 640
_NPAD = _RPT * 16
_NACC = _NPAD + 8
_ROWBLK = 1000
_NBLK = _N // _ROWBLK


def _sc_body(m_hbm, src_hbm, dst_hbm, out_hbm, src_v, dst_v, rows, rows_b,
             acc, sem, sem_b):
    cid = lax.axis_index("c")
    sid = lax.axis_index("s")
    wid = cid * 16 + sid

    def _zrow(i, _):
        for j in range(8):
            rows[i, pl.ds(j * 16, 16)] = jnp.zeros((16,), jnp.float32)
        return 0

    lax.fori_loop(0, _CHUNK, _zrow, 0)
    for t in range(_RPT // _CHUNK):
        pltpu.sync_copy(rows, acc.at[pl.ds(sid * _RPT + t * _CHUNK, _CHUNK)])
    plsc.subcore_barrier()

    for p in range(_NCHUNK // _PHCHUNK):
        pltpu.sync_copy(src_hbm.at[wid, pl.ds(p * _PHCHUNK, _PHCHUNK)], src_v)
        pltpu.sync_copy(dst_hbm.at[wid, pl.ds(p * _PHCHUNK, _PHCHUNK)], dst_v)
        pltpu.async_copy(m_hbm.at[src_v.at[0]], rows, sem)
        pltpu.async_copy(m_hbm.at[src_v.at[1]], rows_b, sem_b)

        def _edge2(k, _):
            ja = 2 * k
            pltpu.make_async_copy(m_hbm.at[src_v.at[ja]], rows, sem).wait()
            pltpu.sync_copy(rows, acc.at[dst_v.at[ja]], add=True)
            pltpu.async_copy(m_hbm.at[src_v.at[ja + 2]], rows, sem)
            pltpu.make_async_copy(
                m_hbm.at[src_v.at[ja + 1]], rows_b, sem_b).wait()
            pltpu.sync_copy(rows_b, acc.at[dst_v.at[ja + 1]], add=True)
            pltpu.async_copy(m_hbm.at[src_v.at[ja + 3]], rows_b, sem_b)
            return 0

        lax.fori_loop(0, _PHCHUNK // 2 - 1, _edge2, 0)
        ja = _PHCHUNK - 2
        pltpu.make_async_copy(m_hbm.at[src_v.at[ja]], rows, sem).wait()
        pltpu.sync_copy(rows, acc.at[dst_v.at[ja]], add=True)
        pltpu.make_async_copy(m_hbm.at[src_v.at[ja + 1]], rows_b, sem_b).wait()
        pltpu.sync_copy(rows_b, acc.at[dst_v.at[ja + 1]], add=True)
    plsc.subcore_barrier()

    pltpu.sync_copy(
        acc.at[pl.ds(sid * _RPT, _RPT)],
        out_hbm.at[cid, pl.ds(sid * _RPT, _RPT)],
    )


@functools.lru_cache(maxsize=1)
def _sc_scatter_fn():
    return functools.partial(
        pl.kernel,
        out_type=jax.ShapeDtypeStruct((2, _NPAD, _D), jnp.float32),
        mesh=plsc.VectorSubcoreMesh(core_axis_name="c", subcore_axis_name="s"),
        scratch_types=[
            pltpu.VMEM((_PHCHUNK, _CHUNK), jnp.int32),
            pltpu.VMEM((_PHCHUNK, _CHUNK), jnp.int32),
            pltpu.VMEM((_CHUNK, _D), jnp.float32),
            pltpu.VMEM((_CHUNK, _D), jnp.float32),
            pltpu.VMEM_SHARED((_NACC, _D), jnp.float32),
            pltpu.SemaphoreType.DMA,
            pltpu.SemaphoreType.DMA,
        ],
    )(_sc_body)


def _sc_scatter(m, src_p, dst_p):
    return _sc_scatter_fn()(m, src_p, dst_p)


def _embed_body(x_ref, we_ref, be_ref, w0_ref, o_ref):
    h = jnp.dot(x_ref[...], we_ref[...], preferred_element_type=jnp.float32)
    h = h + be_ref[...]
    o_ref[...] = jnp.dot(h, w0_ref[...], preferred_element_type=jnp.float32)


def _layer_body(p_ref, b_ref, w_ref, o_ref):
    h = jax.nn.relu(p_ref[0] + p_ref[1] + b_ref[...])
    o_ref[...] = jnp.dot(h, w_ref[...], preferred_element_type=jnp.float32)


def _final_body(p_ref, b_ref, pmat_ref, w0_ref, b0_ref, w1_ref, b1_ref,
                w2_ref, b2_ref, o_ref, pooled, counts):
    i = pl.program_id(0)

    @pl.when(i == 0)
    def _init():
        pooled[...] = jnp.zeros_like(pooled)
        counts[...] = jnp.zeros_like(counts)

    h = jax.nn.relu(p_ref[0] + p_ref[1] + b_ref[...])
    pmat_t = pmat_ref[...]
    dn = (((0,), (0,)), ((), ()))
    pooled[...] += lax.dot_general(
        pmat_t, h, dn, preferred_element_type=jnp.float32)
    counts[...] += lax.dot_general(
        pmat_t, jnp.ones((_ROWBLK, _D), jnp.float32), dn,
        preferred_element_type=jnp.float32)

    @pl.when(i == _NBLK - 1)
    def _finish():
        hg = pooled[...] / jnp.maximum(counts[...], 1.0)
        y = jax.nn.relu(
            jnp.dot(hg, w0_ref[...], preferred_element_type=jnp.float32)
            + b0_ref[...])
        y = jax.nn.relu(
            jnp.dot(y, w1_ref[...], preferred_element_type=jnp.float32)
            + b1_ref[...])
        o_ref[...] = (
            jnp.dot(y, w2_ref[...], preferred_element_type=jnp.float32)
            + b2_ref[...])


_full_w = pl.BlockSpec((_D, _D), lambda i: (0, 0))
_row_b = pl.BlockSpec((1, _D), lambda i: (0, 0))
_xblk = pl.BlockSpec((_ROWBLK, _D), lambda i: (i, 0))
_pblk = pl.BlockSpec((2, _ROWBLK, _D), lambda i: (0, i, 0))


def _tc_embed(x, w_emb, b_emb, w0):
    return pl.pallas_call(
        _embed_body,
        grid=(_NBLK,),
        in_specs=[_xblk, _full_w, _row_b, _full_w],
        out_specs=_xblk,
        out_shape=jax.ShapeDtypeStruct((_N, _D), jnp.float32),
    )(x, w_emb, b_emb, w0)


def _tc_layer(parts, b_prev, w):
    return pl.pallas_call(
        _layer_body,
        grid=(_NBLK,),
        in_specs=[_pblk, _row_b, _full_w],
        out_specs=_xblk,
        out_shape=jax.ShapeDtypeStruct((_N, _D), jnp.float32),
    )(parts, b_prev, w)


def _tc_final(parts, b_prev, pmat, w0, b0, w1, b1, w2, b2):
    return pl.pallas_call(
        _final_body,
        grid=(_NBLK,),
        in_specs=[
            _pblk,
            _row_b,
            pl.BlockSpec((_ROWBLK, _G), lambda i: (i, 0)),
            pl.BlockSpec((_D, _D // 2), lambda i: (0, 0)),
            pl.BlockSpec((1, _D // 2), lambda i: (0, 0)),
            pl.BlockSpec((_D // 2, _D // 4), lambda i: (0, 0)),
            pl.BlockSpec((1, _D // 4), lambda i: (0, 0)),
            pl.BlockSpec((_D // 4, _NC), lambda i: (0, 0)),
            pl.BlockSpec((1, _NC), lambda i: (0, 0)),
        ],
        out_specs=pl.BlockSpec((_G, _NC), lambda i: (0, 0)),
        out_shape=jax.ShapeDtypeStruct((_G, _NC), jnp.float32),
        scratch_shapes=[
            pltpu.VMEM((_G, _D), jnp.float32),
            pltpu.VMEM((_G, _D), jnp.float32),
        ],
    )(parts, b_prev, pmat, w0, b0, w1, b1, w2, b2)


def kernel(x, edge_index, batch, W_emb, b_emb, W_gcn, b_gcn,
           W_mlp0, b_mlp0, W_mlp1, b_mlp1, W_mlp2, b_mlp2):
    src = edge_index[0]
    dst = edge_index[1]
    pad = _EP - _E
    src_p = jnp.concatenate(
        [src, jnp.zeros((pad,), jnp.int32)]).reshape(_NTILES, _NCHUNK, _CHUNK)
    dst_p = jnp.concatenate(
        [dst, jnp.full((pad,), _NPAD, jnp.int32)]).reshape(_NTILES, _NCHUNK, _CHUNK)
    pmat = (batch[:, None] == jnp.arange(_G, dtype=batch.dtype)[None, :]
            ).astype(jnp.float32)

    b_emb2 = b_emb.reshape(1, _D)
    m = _tc_embed(x, W_emb, b_emb2, W_gcn[0])
    for l in range(_NUM_GCN):
        parts = _sc_scatter(m, src_p, dst_p)
        bl = b_gcn[l].reshape(1, _D)
        if l + 1 < _NUM_GCN:
            m = _tc_layer(parts, bl, W_gcn[l + 1])
        else:
            logits = _tc_final(parts, bl, pmat,
                               W_mlp0, b_mlp0.reshape(1, -1),
                               W_mlp1, b_mlp1.reshape(1, -1),
                               W_mlp2, b_mlp2.reshape(1, -1))
    return logits
